# baseline (speedup 1.0000x reference)
import functools

import jax
import jax.numpy as jnp
from jax.experimental import pallas as pl
from jax.experimental.pallas import tpu as pltpu

NEG_FILL = -100.0
BELOW = -200.0
ABOVE = 2.0
INV_TEMP = 1.0 / 0.07
UNROLL = 16


def _l2n(x):
    nrm = jnp.sqrt(jnp.sum(x * x, axis=-1, keepdims=True))
    return x / jnp.maximum(nrm, 1e-12)


def _proj_normed(x_f32, w1_ref, b1_ref, w2_ref, b2_ref):
    xn = _l2n(x_f32).astype(jnp.bfloat16)
    h = jnp.dot(xn, w1_ref[...], preferred_element_type=jnp.float32) + b1_ref[...]
    h = jnp.maximum(h, 0.0).astype(jnp.bfloat16)
    p = jnp.dot(h, w2_ref[...], preferred_element_type=jnp.float32) + b2_ref[...]
    return _l2n(p)


def _extract_step(masked, t, thr, k, n_pos):
    m = jnp.max(jnp.where(masked < t, masked, BELOW), axis=-1, keepdims=True)
    thr = jnp.where(k == n_pos - 1, m, thr)
    return m, thr


def _fused_kernel(nlab, pin_ref, tgt_ref, w1_ref, b1_ref, w2_ref, b2_ref,
                  out_ref, ln_ref, sim_ref):
    i = pl.program_id(0)
    lblk = ln_ref.shape[0] // nlab

    pn = _proj_normed(pin_ref[...], w1_ref, b1_ref, w2_ref,
                      b2_ref).astype(jnp.bfloat16)
    sim_ref[i % 2] = jax.lax.dot_general(
        pn, ln_ref[...], (((1,), (1,)), ((), ())),
        preferred_element_type=jnp.float32)

    sim = sim_ref[(i + 1) % 2]
    tgt = tgt_ref[...]
    pos = tgt > 0
    n_pos = jnp.sum(tgt, axis=-1, keepdims=True)
    masked = jnp.where(pos, NEG_FILL, sim)
    s = sim * INV_TEMP
    es = jnp.exp(s)
    sum_pos_s = jnp.sum(jnp.where(pos, s, 0.0), axis=-1, keepdims=True)
    max_np = jnp.max(n_pos)

    t = jnp.full_like(n_pos, ABOVE, dtype=jnp.float32)
    thr = jnp.full_like(n_pos, BELOW, dtype=jnp.float32)
    for k in range(UNROLL):
        t, thr = _extract_step(masked, t, thr, k, n_pos)

    @pl.when(i < nlab)
    def _store_labels():
        ln_ref[pl.ds(jnp.minimum(i, nlab - 1) * lblk, lblk), :] = pn

    def cond(carry):
        return carry[0] < max_np

    def body(carry):
        k, t, thr = carry
        m, thr = _extract_step(masked, t, thr, k, n_pos)
        return k + 1, m, thr

    _, _, thr = jax.lax.while_loop(cond, body, (jnp.int32(UNROLL), t, thr))

    sel = pos | (masked >= thr)
    denom = jnp.sum(jnp.where(sel, es, 0.0), axis=-1, keepdims=True)
    loss_rows = jnp.log(denom) - sum_pos_s / n_pos.astype(jnp.float32)
    out_ref[...] = jnp.zeros((1, 1, 128), jnp.float32) + jnp.sum(loss_rows)


def _full(shape):
    return pl.BlockSpec(shape, lambda *_: tuple(0 for _ in shape))


def kernel(text_embeddings, label_embeddings, target_labels, W1, b1, W2, b2):
    B, D = text_embeddings.shape
    L = label_embeddings.shape[0]
    H = W1.shape[1]
    blk = 512 if B % 512 == 0 and L % 512 == 0 else None
    if blk is None:
        blk = L if B % L == 0 else B
    nblk = B // blk
    nlab = L // blk
    nsteps = nlab + nblk + 1

    proj_in = jnp.concatenate([label_embeddings, text_embeddings], axis=0)
    w1b = W1.astype(jnp.bfloat16)
    w2b = W2.astype(jnp.bfloat16)
    b1r = b1.reshape(1, H)
    b2r = b2.reshape(1, D)

    partials = pl.pallas_call(
        functools.partial(_fused_kernel, nlab),
        grid=(nsteps,),
        in_specs=[
            pl.BlockSpec((blk, D),
                         lambda i: (jnp.minimum(i, nlab + nblk - 1), 0)),
            pl.BlockSpec((blk, L),
                         lambda i: (jnp.clip(i - nlab - 1, 0, nblk - 1), 0)),
            _full((D, H)), _full((1, H)), _full((H, D)), _full((1, D)),
        ],
        out_specs=pl.BlockSpec(
            (1, 1, 128), lambda i: (jnp.clip(i - nlab - 1, 0, nblk - 1), 0, 0)),
        out_shape=jax.ShapeDtypeStruct((nblk, 1, 128), jnp.float32),
        scratch_shapes=[
            pltpu.VMEM((L, D), jnp.bfloat16),
            pltpu.VMEM((2, blk, L), jnp.float32),
        ],
        compiler_params=pltpu.CompilerParams(
            dimension_semantics=("arbitrary",),
            vmem_limit_bytes=52 * 1024 * 1024,
        ),
        name="tla_loss_fused",
    )(proj_in, target_labels, w1b, b1r, w2b, b2r)

    return jnp.sum(partials[:, 0, 0]) / B


# speedup vs baseline: 1.2396x; 1.2396x over previous
import functools

import jax
import jax.numpy as jnp
from jax.experimental import pallas as pl
from jax.experimental.pallas import tpu as pltpu

NEG_FILL = -100.0
BELOW = -200.0
ABOVE = 2.0
INV_TEMP = 1.0 / 0.07


def _l2n(x):
    nrm = jnp.sqrt(jnp.sum(x * x, axis=-1, keepdims=True))
    return x / jnp.maximum(nrm, 1e-12)


def _proj_normed(x_f32, w1_ref, b1_ref, w2_ref, b2_ref):
    xn = _l2n(x_f32).astype(jnp.bfloat16)
    h = jnp.dot(xn, w1_ref[...], preferred_element_type=jnp.float32) + b1_ref[...]
    h = jnp.maximum(h, 0.0).astype(jnp.bfloat16)
    p = jnp.dot(h, w2_ref[...], preferred_element_type=jnp.float32) + b2_ref[...]
    return _l2n(p)


def _fused_kernel(nlab, lab_ref, txt_ref, tgt_ref, w1_ref, b1_ref, w2_ref,
                  b2_ref, out_ref, ln_ref):
    i = pl.program_id(0)
    lblk = lab_ref.shape[0]

    @pl.when(i < nlab)
    def _label_phase():
        ln = _proj_normed(lab_ref[...], w1_ref, b1_ref, w2_ref, b2_ref)
        ln_ref[pl.ds(i * lblk, lblk), :] = ln.astype(jnp.bfloat16)

    @pl.when(i >= nlab)
    def _text_phase():
        pn = _proj_normed(txt_ref[...], w1_ref, b1_ref, w2_ref,
                          b2_ref).astype(jnp.bfloat16)
        sim = jax.lax.dot_general(pn, ln_ref[...], (((1,), (1,)), ((), ())),
                                  preferred_element_type=jnp.float32)

        tgt = tgt_ref[...]
        pos = tgt > 0
        n_pos = jnp.sum(tgt, axis=-1, keepdims=True)
        masked = jnp.where(pos, NEG_FILL, sim)

        max_np = jnp.max(n_pos)
        t0 = jnp.full_like(n_pos, ABOVE, dtype=jnp.float32)
        thr0 = jnp.full_like(n_pos, BELOW, dtype=jnp.float32)

        nm1 = n_pos - 1

        def cond(carry):
            return carry[0] < max_np

        def body(carry):
            k, t, thr = carry
            m = jnp.max(jnp.where(masked < t, masked, BELOW), axis=-1,
                        keepdims=True)
            thr = jnp.where(k == nm1, m, thr)
            m2 = jnp.max(jnp.where(masked < m, masked, BELOW), axis=-1,
                         keepdims=True)
            thr = jnp.where(k + 1 == nm1, m2, thr)
            return k + 2, m2, thr

        _, _, thr = jax.lax.while_loop(cond, body, (jnp.int32(0), t0, thr0))

        s = sim * INV_TEMP
        es = jnp.exp(s)
        sel = pos | (masked >= thr)
        denom = jnp.sum(jnp.where(sel, es, 0.0), axis=-1, keepdims=True)
        sum_pos_s = jnp.sum(jnp.where(pos, s, 0.0), axis=-1, keepdims=True)
        loss_rows = jnp.log(denom) - sum_pos_s / n_pos.astype(jnp.float32)
        out_ref[...] = jnp.zeros((1, 1, 128), jnp.float32) + jnp.sum(loss_rows)


def _full(shape):
    return pl.BlockSpec(shape, lambda *_: tuple(0 for _ in shape))


def kernel(text_embeddings, label_embeddings, target_labels, W1, b1, W2, b2):
    B, D = text_embeddings.shape
    L = label_embeddings.shape[0]
    H = W1.shape[1]
    blk = 512 if B % 512 == 0 else B
    lblk = L // 2 if L % 2 == 0 else L
    nblk = B // blk
    nlab = L // lblk

    w1b = W1.astype(jnp.bfloat16)
    w2b = W2.astype(jnp.bfloat16)
    b1r = b1.reshape(1, H)
    b2r = b2.reshape(1, D)

    partials = pl.pallas_call(
        functools.partial(_fused_kernel, nlab),
        grid=(nblk + nlab,),
        in_specs=[
            pl.BlockSpec((lblk, D), lambda i: (jnp.minimum(i, nlab - 1), 0)),
            pl.BlockSpec((blk, D), lambda i: (jnp.maximum(i - nlab, 0), 0)),
            pl.BlockSpec((blk, L), lambda i: (jnp.maximum(i - nlab, 0), 0)),
            _full((D, H)), _full((1, H)), _full((H, D)), _full((1, D)),
        ],
        out_specs=pl.BlockSpec((1, 1, 128),
                               lambda i: (jnp.maximum(i - nlab, 0), 0, 0)),
        out_shape=jax.ShapeDtypeStruct((nblk, 1, 128), jnp.float32),
        scratch_shapes=[pltpu.VMEM((L, D), jnp.bfloat16)],
        compiler_params=pltpu.CompilerParams(
            dimension_semantics=("arbitrary",),
            vmem_limit_bytes=52 * 1024 * 1024,
        ),
        name="tla_loss_fused",
    )(label_embeddings, text_embeddings, target_labels, w1b, b1r, w2b, b2r)

    return jnp.sum(partials[:, 0, 0]) / B


# speedup vs baseline: 1.2532x; 1.0109x over previous
import functools

import jax
import jax.numpy as jnp
from jax.experimental import pallas as pl
from jax.experimental.pallas import tpu as pltpu

NEG_FILL = -100.0
BELOW = -200.0
ABOVE = 2.0
INV_TEMP = 1.0 / 0.07


def _l2n(x):
    nrm = jnp.sqrt(jnp.sum(x * x, axis=-1, keepdims=True))
    return x / jnp.maximum(nrm, 1e-12)


def _proj_normed(x_f32, w1_ref, b1_ref, w2_ref, b2_ref):
    xn = _l2n(x_f32).astype(jnp.bfloat16)
    h = jnp.dot(xn, w1_ref[...], preferred_element_type=jnp.float32) + b1_ref[...]
    h = jnp.maximum(h, 0.0).astype(jnp.bfloat16)
    p = jnp.dot(h, w2_ref[...], preferred_element_type=jnp.float32) + b2_ref[...]
    return _l2n(p)


def _fused_kernel(nlab, lab_ref, txt_ref, tgt_ref, w1_ref, b1_ref, w2_ref,
                  b2_ref, out_ref, ln_ref):
    i = pl.program_id(0)
    lblk = lab_ref.shape[0]

    @pl.when(i < nlab)
    def _label_phase():
        ln = _proj_normed(lab_ref[...], w1_ref, b1_ref, w2_ref, b2_ref)
        ln_ref[pl.ds(i * lblk, lblk), :] = ln.astype(jnp.bfloat16)

    @pl.when(i >= nlab)
    def _text_phase():
        pn = _proj_normed(txt_ref[...], w1_ref, b1_ref, w2_ref,
                          b2_ref).astype(jnp.bfloat16)
        sim = jax.lax.dot_general(pn, ln_ref[...], (((1,), (1,)), ((), ())),
                                  preferred_element_type=jnp.float32)

        tgt = tgt_ref[...]
        pos = tgt > 0
        n_pos = jnp.sum(tgt, axis=-1, keepdims=True)
        masked = jnp.where(pos, NEG_FILL, sim)

        max_np = jnp.max(n_pos)
        t0 = jnp.full_like(n_pos, ABOVE, dtype=jnp.float32)
        thr0 = jnp.full_like(n_pos, BELOW, dtype=jnp.float32)

        nm1 = n_pos - 1

        def cond(carry):
            return carry[0] < max_np

        def body(carry):
            k, t, thr = carry
            for d in range(4):
                m = jnp.max(jnp.where(masked < t, masked, BELOW), axis=-1,
                            keepdims=True)
                thr = jnp.where(k + d == nm1, m, thr)
                t = m
            return k + 4, t, thr

        _, _, thr = jax.lax.while_loop(cond, body, (jnp.int32(0), t0, thr0))

        s = sim * INV_TEMP
        es = jnp.exp(s)
        sel = pos | (masked >= thr)
        denom = jnp.sum(jnp.where(sel, es, 0.0), axis=-1, keepdims=True)
        sum_pos_s = jnp.sum(jnp.where(pos, s, 0.0), axis=-1, keepdims=True)
        loss_rows = jnp.log(denom) - sum_pos_s / n_pos.astype(jnp.float32)
        out_ref[...] = jnp.zeros((1, 1, 128), jnp.float32) + jnp.sum(loss_rows)


def _full(shape):
    return pl.BlockSpec(shape, lambda *_: tuple(0 for _ in shape))


def kernel(text_embeddings, label_embeddings, target_labels, W1, b1, W2, b2):
    B, D = text_embeddings.shape
    L = label_embeddings.shape[0]
    H = W1.shape[1]
    blk = 512 if B % 512 == 0 else B
    lblk = L // 2 if L % 2 == 0 else L
    nblk = B // blk
    nlab = L // lblk

    w1b = W1.astype(jnp.bfloat16)
    w2b = W2.astype(jnp.bfloat16)
    b1r = b1.reshape(1, H)
    b2r = b2.reshape(1, D)

    partials = pl.pallas_call(
        functools.partial(_fused_kernel, nlab),
        grid=(nblk + nlab,),
        in_specs=[
            pl.BlockSpec((lblk, D), lambda i: (jnp.minimum(i, nlab - 1), 0)),
            pl.BlockSpec((blk, D), lambda i: (jnp.maximum(i - nlab, 0), 0)),
            pl.BlockSpec((blk, L), lambda i: (jnp.maximum(i - nlab, 0), 0)),
            _full((D, H)), _full((1, H)), _full((H, D)), _full((1, D)),
        ],
        out_specs=pl.BlockSpec((1, 1, 128),
                               lambda i: (jnp.maximum(i - nlab, 0), 0, 0)),
        out_shape=jax.ShapeDtypeStruct((nblk, 1, 128), jnp.float32),
        scratch_shapes=[pltpu.VMEM((L, D), jnp.bfloat16)],
        compiler_params=pltpu.CompilerParams(
            dimension_semantics=("arbitrary",),
            vmem_limit_bytes=52 * 1024 * 1024,
        ),
        name="tla_loss_fused",
    )(label_embeddings, text_embeddings, target_labels, w1b, b1r, w2b, b2r)

    return jnp.sum(partials[:, 0, 0]) / B
